# baseline (speedup 1.0000x reference)
import jax
import jax.numpy as jnp
from jax import lax
from jax.experimental import pallas as pl
from jax.experimental.pallas import tpu as pltpu
from jax.experimental.pallas import tpu_sc as plsc

BATCH = 4
MAX_LEN = 2048
EMBED = 1024
NC, NS, L = 2, 16, 16
NW = NC * NS
T_PER_W = MAX_LEN // NW
CT = 8
NCHUNK = T_PER_W // CT
NBUF = 3
VREGS_PER_ROW = EMBED // L


def _body(seq_hbm, tok_hbm, pos_hbm, out_hbm, idx_v,
          rows0, rows1, rows2, pos0, pos1, pos2,
          gsem0, gsem1, gsem2, ssem0, ssem1, ssem2):
    wid = lax.axis_index("s") * NC + lax.axis_index("c")
    tw0 = wid * T_PER_W
    for b in range(BATCH):
        pltpu.sync_copy(seq_hbm.at[b, pl.ds(tw0, T_PER_W)], idx_v.at[b])

    rows = [rows0, rows1, rows2]
    pos = [pos0, pos1, pos2]
    gsem = [gsem0, gsem1, gsem2]
    ssem = [ssem0, ssem1, ssem2]

    def start_unit(c):
        buf = c % NBUF
        t0 = tw0 + c * CT
        descs = [pltpu.async_copy(pos_hbm.at[pl.ds(t0, CT)], pos[buf], gsem[buf])]
        for b in range(BATCH):
            descs.append(pltpu.async_copy(
                tok_hbm.at[idx_v.at[b, pl.ds(c * CT, CT)]],
                rows[buf].at[b], gsem[buf]))
        return descs

    pend_g = {c: start_unit(c) for c in range(NBUF - 1)}
    pend_s = {}
    for c in range(NCHUNK):
        buf = c % NBUF
        nxt = c + NBUF - 1
        if nxt < NCHUNK:
            for d in pend_s.pop(nxt % NBUF, ()):
                d.wait()
            pend_g[nxt] = start_unit(nxt)
        for d in pend_g.pop(c):
            d.wait()

        t0 = tw0 + c * CT
        sdescs = []
        for bb in range(BATCH):

            def add_j(j, carry, _buf=buf, _bb=bb):
                sl = pl.ds(j * L, L)
                for r in range(CT):
                    rows[_buf][_bb, r, sl] = rows[_buf][_bb, r, sl] + pos[_buf][r, sl]
                return carry

            lax.fori_loop(0, VREGS_PER_ROW, add_j, 0)
            sdescs.append(pltpu.async_copy(
                rows[buf].at[bb], out_hbm.at[bb, pl.ds(t0, CT)], ssem[buf]))
        pend_s[buf] = sdescs
    for descs in pend_s.values():
        for d in descs:
            d.wait()


def kernel(sequence, token_weight, position_weight):
    mesh = plsc.VectorSubcoreMesh(core_axis_name="c", subcore_axis_name="s")
    f = pl.kernel(
        _body,
        out_type=jax.ShapeDtypeStruct((BATCH, MAX_LEN, EMBED), jnp.float32),
        mesh=mesh,
        scratch_types=[
            pltpu.VMEM((BATCH, T_PER_W), jnp.int32),
            pltpu.VMEM((BATCH, CT, EMBED), jnp.float32),
            pltpu.VMEM((BATCH, CT, EMBED), jnp.float32),
            pltpu.VMEM((BATCH, CT, EMBED), jnp.float32),
            pltpu.VMEM((CT, EMBED), jnp.float32),
            pltpu.VMEM((CT, EMBED), jnp.float32),
            pltpu.VMEM((CT, EMBED), jnp.float32),
            pltpu.SemaphoreType.DMA,
            pltpu.SemaphoreType.DMA,
            pltpu.SemaphoreType.DMA,
            pltpu.SemaphoreType.DMA,
            pltpu.SemaphoreType.DMA,
            pltpu.SemaphoreType.DMA,
        ],
    )
    return f(sequence, token_weight, position_weight)


# speedup vs baseline: 1.0856x; 1.0856x over previous
import jax
import jax.numpy as jnp
from jax import lax
from jax.experimental import pallas as pl
from jax.experimental.pallas import tpu as pltpu
from jax.experimental.pallas import tpu_sc as plsc

BATCH = 4
MAX_LEN = 2048
EMBED = 1024
NC, NS, L = 2, 16, 16
NW = NC * NS
T_PER_W = MAX_LEN // NW
CT = 8
NCHUNK = T_PER_W // CT
NBUF = 3
VREGS_PER_ROW = EMBED // L


def _body(seq_hbm, tok_hbm, pos_hbm, out_hbm, idx_v,
          rows0, rows1, rows2, pos0, pos1, pos2,
          gsem0, gsem1, gsem2, ssem0, ssem1, ssem2):
    wid = lax.axis_index("s") * NC + lax.axis_index("c")
    tw0 = wid * T_PER_W
    for b in range(BATCH):
        pltpu.sync_copy(seq_hbm.at[b, pl.ds(tw0, T_PER_W)], idx_v.at[b])

    rows = [rows0, rows1, rows2]
    pos = [pos0, pos1, pos2]
    gsem = [gsem0, gsem1, gsem2]
    ssem = [ssem0, ssem1, ssem2]

    def start_unit(c):
        buf = c % NBUF
        t0 = tw0 + c * CT
        descs = [pltpu.async_copy(pos_hbm.at[pl.ds(t0, CT)], pos[buf], gsem[buf])]
        for b in range(BATCH):
            descs.append(pltpu.async_copy(
                tok_hbm.at[idx_v.at[b, pl.ds(c * CT, CT)]],
                rows[buf].at[b], gsem[buf]))
        return descs

    pend_g = {c: start_unit(c) for c in range(NBUF - 1)}
    pend_s = {}
    for c in range(NCHUNK):
        buf = c % NBUF
        nxt = c + NBUF - 1
        if nxt < NCHUNK:
            for d in pend_s.pop(nxt % NBUF, ()):
                d.wait()
            pend_g[nxt] = start_unit(nxt)
        gdescs = pend_g.pop(c)

        t0 = tw0 + c * CT
        sdescs = []
        for pr in range(BATCH // 2):
            if pr == 0:
                gdescs[0].wait()
            for b in (2 * pr, 2 * pr + 1):
                gdescs[1 + b].wait()

            def add_j(j, carry, _buf=buf, _pr=pr):
                sl = pl.ds(j * L, L)
                for r in range(CT):
                    p = pos[_buf][r, sl]
                    for b in (2 * _pr, 2 * _pr + 1):
                        rows[_buf][b, r, sl] = rows[_buf][b, r, sl] + p
                return carry

            lax.fori_loop(0, VREGS_PER_ROW, add_j, 0)
            for b in (2 * pr, 2 * pr + 1):
                sdescs.append(pltpu.async_copy(
                    rows[buf].at[b], out_hbm.at[b, pl.ds(t0, CT)], ssem[buf]))
        pend_s[buf] = sdescs
    for descs in pend_s.values():
        for d in descs:
            d.wait()


def kernel(sequence, token_weight, position_weight):
    mesh = plsc.VectorSubcoreMesh(core_axis_name="c", subcore_axis_name="s")
    f = pl.kernel(
        _body,
        out_type=jax.ShapeDtypeStruct((BATCH, MAX_LEN, EMBED), jnp.float32),
        mesh=mesh,
        scratch_types=[
            pltpu.VMEM((BATCH, T_PER_W), jnp.int32),
            pltpu.VMEM((BATCH, CT, EMBED), jnp.float32),
            pltpu.VMEM((BATCH, CT, EMBED), jnp.float32),
            pltpu.VMEM((BATCH, CT, EMBED), jnp.float32),
            pltpu.VMEM((CT, EMBED), jnp.float32),
            pltpu.VMEM((CT, EMBED), jnp.float32),
            pltpu.VMEM((CT, EMBED), jnp.float32),
            pltpu.SemaphoreType.DMA,
            pltpu.SemaphoreType.DMA,
            pltpu.SemaphoreType.DMA,
            pltpu.SemaphoreType.DMA,
            pltpu.SemaphoreType.DMA,
            pltpu.SemaphoreType.DMA,
        ],
    )
    return f(sequence, token_weight, position_weight)
